# baseline (speedup 1.0000x reference)
import functools

import jax
import jax.numpy as jnp
from jax import lax
from jax.experimental import pallas as pl
from jax.experimental.pallas import tpu as pltpu
from jax.experimental.pallas import tpu_sc as plsc

_LANES = 128
_PB = 2


def _pad_body(x_ref, o_ref):
    S = x_ref.shape[1]
    z = jnp.zeros((x_ref.shape[0], _LANES - S), jnp.int32)
    o_ref[...] = jnp.concatenate([x_ref[...], z], axis=1)


def _pad_x(x):
    Bt, S = x.shape
    BLK = 2048
    return pl.pallas_call(
        _pad_body,
        grid=(Bt // BLK,),
        in_specs=[pl.BlockSpec((BLK, S), lambda i: (i, 0))],
        out_specs=pl.BlockSpec((BLK, _LANES), lambda i: (i, 0)),
        out_shape=jax.ShapeDtypeStruct((Bt, _LANES), jnp.int32),
    )(x)


def _sc_gather(table, xpad, Bt, S, D, SP):
    V, _ = table.shape
    info = plsc.get_sparse_core_info()
    NC, NS = info.num_cores, info.num_subcores
    NW = NC * NS
    assert Bt % NW == 0
    b_per_w = Bt // NW
    assert b_per_w % _PB == 0
    n_super = b_per_w // _PB
    assert n_super % 2 == 0
    rows_per_super = _PB * SP

    mesh = plsc.VectorSubcoreMesh(core_axis_name="c", subcore_axis_name="s")

    @functools.partial(
        pl.kernel,
        mesh=mesh,
        compiler_params=pltpu.CompilerParams(use_tc_tiling_on_sc=False),
        out_type=jax.ShapeDtypeStruct((Bt * SP, _LANES), jnp.float32),
        scratch_types=[
            pltpu.VMEM((b_per_w, _LANES), jnp.int32),
            pltpu.VMEM((rows_per_super, D), jnp.float32),
            pltpu.VMEM((rows_per_super, D), jnp.float32),
            pltpu.VMEM((rows_per_super, _LANES), jnp.float32),
            pltpu.VMEM((rows_per_super, _LANES), jnp.float32),
            pltpu.SemaphoreType.DMA,
            pltpu.SemaphoreType.DMA,
            pltpu.SemaphoreType.DMA,
            pltpu.SemaphoreType.DMA,
        ],
    )
    def k(table_hbm, x_hbm, out_hbm, idx_v, cb0, cb1, rows0, rows1,
          g0, g1, o0, o1):
        wid = lax.axis_index("s") * NC + lax.axis_index("c")
        wbase = wid * b_per_w
        pltpu.sync_copy(x_hbm.at[pl.ds(wbase, b_per_w)], idx_v)
        cbufs = (cb0, cb1)
        bufs = (rows0, rows1)
        gsems = (g0, g1)
        osems = (o0, o1)

        zv = jnp.zeros((16,), jnp.float32)

        def zbody(r, carry):
            for buf in bufs:
                for c in range(D, _LANES, 16):
                    buf[r, pl.ds(c, 16)] = zv
            return carry

        lax.fori_loop(0, rows_per_super, zbody, 0)

        def superchunk(sc, half, first):
            cbuf = cbufs[half]
            buf, gsem, osem = bufs[half], gsems[half], osems[half]
            @pl.when(jnp.logical_not(first))
            def _():
                pltpu.make_async_copy(
                    buf,
                    out_hbm.at[pl.ds(0, rows_per_super)],
                    osem,
                ).wait()
            descs = []
            for bb in range(_PB):
                d = pltpu.async_copy(
                    table_hbm.at[idx_v.at[sc * _PB + bb, pl.ds(0, SP)]],
                    cbuf.at[pl.ds(bb * SP, SP)],
                    gsem,
                )
                descs.append(d)
            for d in descs:
                d.wait()

            def rbody(r4, carry):
                r = r4 * 4
                for dr in range(4):
                    for c in range(0, D, 16):
                        buf[r + dr, pl.ds(c, 16)] = cbuf[r + dr, pl.ds(c, 16)]
                return carry

            lax.fori_loop(0, rows_per_super // 4, rbody, 0)
            pltpu.async_copy(
                buf,
                out_hbm.at[pl.ds((wbase + sc * _PB) * SP, rows_per_super)],
                osem,
            )

        def body(i2, carry):
            superchunk(2 * i2, 0, i2 == 0)
            superchunk(2 * i2 + 1, 1, i2 == 0)
            return carry

        lax.fori_loop(0, n_super // 2, body, 0)
        for half in range(2):
            pltpu.make_async_copy(
                bufs[half],
                out_hbm.at[pl.ds(0, rows_per_super)],
                osems[half],
            ).wait()

    return k(table, xpad)


_SQRT_HALF = 0.7071067811865476


def _make_adapter_body(BB, S, D, SP):
    def body(x_ref, w_ref, b_ref, o_ref):
        v = x_ref[...]
        h = jnp.dot(v, w_ref[...], preferred_element_type=jnp.float32)
        h = h + b_ref[...]
        g = h * 0.5 * (1.0 + lax.erf(h * _SQRT_HALF))
        o_ref[...] = g.reshape(BB, SP, _LANES)[:, :S, :D]

    return body


def _tc_adapter(G, W128, b128, Bt, S, D, SP):
    BB = 128
    assert Bt % BB == 0
    return pl.pallas_call(
        _make_adapter_body(BB, S, D, SP),
        grid=(Bt // BB,),
        in_specs=[
            pl.BlockSpec((BB * SP, _LANES), lambda i: (i, 0)),
            pl.BlockSpec((_LANES, _LANES), lambda i: (0, 0)),
            pl.BlockSpec((1, _LANES), lambda i: (0, 0)),
        ],
        out_specs=pl.BlockSpec((BB, S, D), lambda i: (i, 0, 0)),
        out_shape=jax.ShapeDtypeStruct((Bt, S, D), jnp.float32),
    )(G, W128, b128)


def kernel(x, table, W, b):
    Bt, S = x.shape
    V, D = table.shape
    SP = ((S + 7) // 8) * 8
    xpad = _pad_x(x.astype(jnp.int32))
    G = _sc_gather(table, xpad, Bt, S, D, SP)
    W128 = jnp.zeros((_LANES, _LANES), jnp.float32).at[:D, :D].set(W.T)
    b128 = jnp.zeros((1, _LANES), jnp.float32).at[0, :D].set(b)
    return _tc_adapter(G, W128, b128, Bt, S, D, SP)
